# baseline (speedup 1.0000x reference)
import functools

import jax
import jax.numpy as jnp
from jax import lax
from jax.experimental import pallas as pl
from jax.experimental.pallas import tpu as pltpu
from jax.experimental.pallas import tpu_sc as plsc

_B, _L = 16384, 200

_INFO = plsc.get_sparse_core_info()
_NC = _INFO.num_cores
_NS = _INFO.num_subcores
_LANES = _INFO.num_lanes
_NW = _NC * _NS

_SC_COLS = 8192
_TC_COLS = _B - _SC_COLS
_COLS_W = _SC_COLS // _NW
assert _COLS_W * _NW == _SC_COLS and _COLS_W % _LANES == 0
_TC_BLK = 1024
assert _TC_COLS % _TC_BLK == 0 and _SC_COLS % _TC_BLK == 0


def _mask_eq_acc(x, acc):
    input_mask = x != 0
    random_mask = jnp.zeros(x.shape, jnp.bool_)
    mask_no_alter = jnp.logical_and(random_mask, input_mask)
    mask_alter = jnp.logical_and(random_mask, input_mask)
    eq = mask_no_alter == mask_alter
    return jnp.logical_and(acc, eq)


def _make_sc_kernel():
    mesh = plsc.VectorSubcoreMesh(core_axis_name="c", subcore_axis_name="s")

    @functools.partial(
        pl.kernel,
        mesh=mesh,
        out_type=jax.ShapeDtypeStruct((_NW, _LANES), jnp.int32),
        scratch_types=[
            pltpu.VMEM((_L, _COLS_W), jnp.int32),
            pltpu.VMEM((_LANES,), jnp.int32),
            pltpu.SemaphoreType.DMA,
        ],
        compiler_params=pltpu.CompilerParams(use_tc_tiling_on_sc=True),
    )
    def sc_masks_equal(tokens_hbm, out_hbm, buf, res, sem):
        wid = lax.axis_index("s") * _NC + lax.axis_index("c")
        base = wid * _COLS_W
        pltpu.async_copy(tokens_hbm.at[:, pl.ds(base, _COLS_W)], buf, sem).wait()

        def step(r, acc):
            for v in range(_COLS_W // _LANES):
                acc = _mask_eq_acc(buf[r, pl.ds(v * _LANES, _LANES)], acc)
            return acc

        acc = lax.fori_loop(0, _L, step, jnp.ones((_LANES,), jnp.bool_))
        res[...] = acc.astype(jnp.int32)
        pltpu.sync_copy(res, out_hbm.at[wid])

    return sc_masks_equal


_SC_MASKS_EQUAL = _make_sc_kernel()


def _tc_body(x_ref, o_ref):
    ok = jnp.all(_mask_eq_acc(x_ref[...], jnp.ones(x_ref.shape, jnp.bool_)))
    ok = ok.astype(jnp.int32)
    i = pl.program_id(0)

    @pl.when(i == 0)
    def _init():
        o_ref[0, 0] = ok

    @pl.when(i != 0)
    def _acc():
        o_ref[0, 0] &= ok


_TC_MASKS_EQUAL = pl.pallas_call(
    _tc_body,
    grid=(_TC_COLS // _TC_BLK,),
    in_specs=[
        pl.BlockSpec((_L, _TC_BLK), lambda i: (0, _SC_COLS // _TC_BLK + i))
    ],
    out_specs=pl.BlockSpec(memory_space=pltpu.SMEM),
    out_shape=jax.ShapeDtypeStruct((1, 1), jnp.int32),
)


def kernel(inputs, table):
    del table
    tokens = inputs.T
    sc_part = _SC_MASKS_EQUAL(tokens)
    tc_part = _TC_MASKS_EQUAL(tokens)
    return jnp.logical_and(jnp.all(sc_part == 1), tc_part[0, 0] == 1)
